# baseline (speedup 1.0000x reference)
import dataclasses
import functools

import jax
import jax.numpy as jnp
from jax import lax
from jax.experimental import pallas as pl
from jax.experimental.pallas import tpu as pltpu
from jax.experimental.pallas import tpu_sc as plsc

B = 16384
D = 128
NC, NS = 2, 16
NW = NC * NS
BPW = B // NW
CH = 128
NCH = BPW // CH
L = 16
_BREV4 = [int(f"{i:04b}"[::-1], 2) for i in range(16)]

_DNUMS = lax.GatherDimensionNumbers(
    offset_dims=(), collapsed_slice_dims=(0,), start_index_map=(0,))


def _take16(x, idx):
    return lax.gather(x, idx[:, None], _DNUMS, (1,),
                      mode=lax.GatherScatterMode.PROMISE_IN_BOUNDS)


_mesh = plsc.VectorSubcoreMesh(core_axis_name="c", subcore_axis_name="s")

_cp = pltpu.CompilerParams()
if "needs_layout_passes" in pltpu.CompilerParams.__dataclass_fields__:
    _cp = dataclasses.replace(_cp, needs_layout_passes=False)


@functools.partial(
    pl.kernel,
    compiler_params=_cp,
    out_type=jax.ShapeDtypeStruct((B,), jnp.float32),
    mesh=_mesh,
    scratch_types=[
        pltpu.VMEM((BPW,), jnp.int32),
        pltpu.VMEM((BPW,), jnp.int32),
        pltpu.VMEM((CH, D), jnp.float32),
        pltpu.VMEM((CH, D), jnp.float32),
        pltpu.VMEM((CH, D), jnp.float32),
        pltpu.VMEM((CH, D), jnp.float32),
        pltpu.VMEM((BPW,), jnp.float32),
        pltpu.SemaphoreType.DMA,
        pltpu.SemaphoreType.DMA,
    ],
)
def _sc_mf(uidx_hbm, iidx_hbm, utab_hbm, itab_hbm, out_hbm,
           idx_u, idx_i, u0, u1, v0, v1, ob, sem0, sem1):
    wid = lax.axis_index("s") * NC + lax.axis_index("c")
    base = wid * BPW
    pltpu.sync_copy(uidx_hbm.at[pl.ds(base, BPW)], idx_u)
    pltpu.sync_copy(iidx_hbm.at[pl.ds(base, BPW)], idx_i)

    bufs = ((u0, v0, sem0), (u1, v1, sem1))
    lanes = lax.iota(jnp.int32, L)

    def start(ck, slot):
        ub, vb, sem = bufs[slot]
        pltpu.async_copy(utab_hbm.at[idx_u.at[pl.ds(ck * CH, CH)]], ub, sem)
        pltpu.async_copy(itab_hbm.at[idx_i.at[pl.ds(ck * CH, CH)]], vb, sem)

    def drain(slot):
        ub, vb, sem = bufs[slot]
        pltpu.make_async_copy(
            utab_hbm.at[idx_u.at[pl.ds(0, CH)]], ub, sem).wait()
        pltpu.make_async_copy(
            itab_hbm.at[idx_i.at[pl.ds(0, CH)]], vb, sem).wait()

    def compute(slot, ck):
        ub, vb, _ = bufs[slot]

        @pl.loop(0, CH // L)
        def _(g):
            rows = g * L + lanes
            bases = [(lanes * 9 + t) & (D - 1) for t in range(L)]
            tots = [jnp.zeros((L,), jnp.float32) for _ in range(4)]
            for j in range(D):
                a, t = divmod(j, L)
                cols = (bases[t] + a * L) & (D - 1)
                cu = plsc.load_gather(ub, [rows, cols])
                cv = plsc.load_gather(vb, [rows, cols])
                tots[j % 4] = tots[j % 4] + cu * cv
            ob[pl.ds(ck * CH + g * L, L)] = (
                (tots[0] + tots[1]) + (tots[2] + tots[3]))

    start(0, 0)
    start(1, 1)

    @pl.loop(0, NCH, step=2)
    def _(ck):
        for b in range(2):
            cur = ck + b
            drain(b)

            @pl.when(cur + 2 < NCH)
            def _():
                start(cur + 2, b)

            compute(b, cur)

    pltpu.sync_copy(ob, out_hbm.at[pl.ds(base, BPW)])


def kernel(x, user_embedding, item_embedding):
    uidx = x[:, 0].astype(jnp.int32)
    iidx = x[:, 1].astype(jnp.int32)
    return _sc_mf(uidx, iidx, user_embedding, item_embedding)


# speedup vs baseline: 1.0037x; 1.0037x over previous
import dataclasses
import functools

import jax
import jax.numpy as jnp
from jax import lax
from jax.experimental import pallas as pl
from jax.experimental.pallas import tpu as pltpu
from jax.experimental.pallas import tpu_sc as plsc

B = 16384
D = 128
NC, NS = 2, 16
NW = NC * NS
BPW = B // NW
CH = 128
NCH = BPW // CH
L = 16
_BREV4 = [int(f"{i:04b}"[::-1], 2) for i in range(16)]

_DNUMS = lax.GatherDimensionNumbers(
    offset_dims=(), collapsed_slice_dims=(0,), start_index_map=(0,))


def _take16(x, idx):
    return lax.gather(x, idx[:, None], _DNUMS, (1,),
                      mode=lax.GatherScatterMode.PROMISE_IN_BOUNDS)


_mesh = plsc.VectorSubcoreMesh(core_axis_name="c", subcore_axis_name="s")

_cp = pltpu.CompilerParams()
if "needs_layout_passes" in pltpu.CompilerParams.__dataclass_fields__:
    _cp = dataclasses.replace(_cp, needs_layout_passes=False)


@functools.partial(
    pl.kernel,
    compiler_params=_cp,
    out_type=jax.ShapeDtypeStruct((B,), jnp.float32),
    mesh=_mesh,
    scratch_types=[
        pltpu.VMEM((BPW,), jnp.int32),
        pltpu.VMEM((BPW,), jnp.int32),
        pltpu.VMEM((CH, D), jnp.float32),
        pltpu.VMEM((CH, D), jnp.float32),
        pltpu.VMEM((CH, D), jnp.float32),
        pltpu.VMEM((CH, D), jnp.float32),
        pltpu.VMEM((BPW,), jnp.float32),
        pltpu.SemaphoreType.DMA,
        pltpu.SemaphoreType.DMA,
    ],
)
def _sc_mf(uidx_hbm, iidx_hbm, utab_hbm, itab_hbm, out_hbm,
           idx_u, idx_i, u0, u1, v0, v1, ob, sem0, sem1):
    wid = lax.axis_index("s") * NC + lax.axis_index("c")
    base = wid * BPW
    pltpu.sync_copy(uidx_hbm.at[pl.ds(base, BPW)], idx_u)
    pltpu.sync_copy(iidx_hbm.at[pl.ds(base, BPW)], idx_i)

    bufs = ((u0, v0, sem0), (u1, v1, sem1))
    lanes = lax.iota(jnp.int32, L)

    def start(ck, slot):
        ub, vb, sem = bufs[slot]
        pltpu.async_copy(utab_hbm.at[idx_u.at[pl.ds(ck * CH, CH)]], ub, sem)
        pltpu.async_copy(itab_hbm.at[idx_i.at[pl.ds(ck * CH, CH)]], vb, sem)

    def drain(slot):
        ub, vb, sem = bufs[slot]
        pltpu.make_async_copy(
            utab_hbm.at[idx_u.at[pl.ds(0, CH)]], ub, sem).wait()
        pltpu.make_async_copy(
            itab_hbm.at[idx_i.at[pl.ds(0, CH)]], vb, sem).wait()

    def compute(slot, ck):
        ub, vb, _ = bufs[slot]

        @pl.loop(0, CH // L)
        def _(g):
            rows = g * L + lanes
            bases = [(lanes * 9 + t) & (D - 1) for t in range(L)]
            tots = [jnp.zeros((L,), jnp.float32) for _ in range(4)]
            for j in range(D):
                a, t = divmod(j, L)
                cols = (bases[t] + a * L) & (D - 1)
                cu = plsc.load_gather(ub, [rows, cols])
                cv = plsc.load_gather(vb, [rows, cols])
                tots[j % 4] = tots[j % 4] + cu * cv
            ob[pl.ds(ck * CH + g * L, L)] = (
                (tots[0] + tots[1]) + (tots[2] + tots[3]))

    start(0, 0)
    start(1, 1)

    @pl.loop(0, NCH, step=2)
    def _(ck):
        for b in range(2):
            cur = ck + b
            drain(b)
            compute(b, cur)

            @pl.when(cur + 2 < NCH)
            def _():
                start(cur + 2, b)

    pltpu.sync_copy(ob, out_hbm.at[pl.ds(base, BPW)])


def kernel(x, user_embedding, item_embedding):
    uidx = x[:, 0].astype(jnp.int32)
    iidx = x[:, 1].astype(jnp.int32)
    return _sc_mf(uidx, iidx, user_embedding, item_embedding)


# speedup vs baseline: 1.6909x; 1.6847x over previous
import dataclasses
import functools

import jax
import jax.numpy as jnp
from jax import lax
from jax.experimental import pallas as pl
from jax.experimental.pallas import tpu as pltpu
from jax.experimental.pallas import tpu_sc as plsc

B = 16384
D = 128
NC, NS = 2, 16
NW = NC * NS
BPW = B // NW
CH = 128
NCH = BPW // CH
L = 16
_BREV4 = [int(f"{i:04b}"[::-1], 2) for i in range(16)]

_DNUMS = lax.GatherDimensionNumbers(
    offset_dims=(), collapsed_slice_dims=(0,), start_index_map=(0,))


def _take16(x, idx):
    return lax.gather(x, idx[:, None], _DNUMS, (1,),
                      mode=lax.GatherScatterMode.PROMISE_IN_BOUNDS)

_mesh = plsc.VectorSubcoreMesh(core_axis_name="c", subcore_axis_name="s")

_cp = pltpu.CompilerParams()


@functools.partial(
    pl.kernel,
    compiler_params=_cp,
    out_type=jax.ShapeDtypeStruct((B,), jnp.float32),
    mesh=_mesh,
    scratch_types=[
        pltpu.VMEM((BPW,), jnp.int32),
        pltpu.VMEM((BPW,), jnp.int32),
        pltpu.VMEM((CH, D), jnp.float32),
        pltpu.VMEM((CH, D), jnp.float32),
        pltpu.VMEM((CH, D), jnp.float32),
        pltpu.VMEM((CH, D), jnp.float32),
        pltpu.VMEM((BPW,), jnp.float32),
        pltpu.VMEM((CH, L), jnp.float32),
        pltpu.SemaphoreType.DMA,
        pltpu.SemaphoreType.DMA,
    ],
)
def _sc_mf(uidx_hbm, iidx_hbm, utab_hbm, itab_hbm, out_hbm,
           idx_u, idx_i, u0, u1, v0, v1, ob, part, sem0, sem1):
    wid = lax.axis_index("s") * NC + lax.axis_index("c")
    base = wid * BPW
    pltpu.sync_copy(uidx_hbm.at[pl.ds(base, BPW)], idx_u)
    pltpu.sync_copy(iidx_hbm.at[pl.ds(base, BPW)], idx_i)

    bufs = ((u0, v0, sem0), (u1, v1, sem1))
    lanes = lax.iota(jnp.int32, L)

    def start(ck, slot):
        ub, vb, sem = bufs[slot]
        cu = pltpu.async_copy(
            utab_hbm.at[idx_u.at[pl.ds(ck * CH, CH)]], ub, sem)
        cv = pltpu.async_copy(
            itab_hbm.at[idx_i.at[pl.ds(ck * CH, CH)]], vb, sem)
        return cu, cv

    def hsum16(rows):
        vs = [rows[_BREV4[i]] for i in range(L)]
        for half in (8, 4, 2, 1):
            idx = lanes ^ half
            mask = (lanes & half) != 0
            nxt = []
            for k in range(0, len(vs), 2):
                a, b = vs[k], vs[k + 1]
                fa = a + _take16(a, idx)
                fb = b + _take16(b, idx)
                nxt.append(
                    jnp.where(mask, _take16(fb, idx),
                              fa))
            vs = nxt
        return vs[0]

    def compute(ck, slot):
        ub, vb, _ = bufs[slot]

        @pl.loop(0, CH)
        def _(r):
            prods = [
                ub[r, pl.ds(j * L, L)] * vb[r, pl.ds(j * L, L)]
                for j in range(D // L)
            ]
            while len(prods) > 1:
                prods = [
                    prods[k] + prods[k + 1]
                    for k in range(0, len(prods), 2)
                ]
            part[r, :] = prods[0]

        @pl.loop(0, CH // L)
        def _(g):
            accs = [part[g * L + i, :] for i in range(L)]
            ob[pl.ds(ck * CH + g * L, L)] = hsum16(accs)

    pending = {0: start(0, 0)}
    for ck in range(NCH):
        if ck + 1 < NCH:
            pending[ck + 1] = start(ck + 1, (ck + 1) % 2)
        for c in pending.pop(ck):
            c.wait()
        compute(ck, ck % 2)

    pltpu.sync_copy(ob, out_hbm.at[pl.ds(base, BPW)])


def kernel(x, user_embedding, item_embedding):
    uidx = x[:, 0].astype(jnp.int32)
    iidx = x[:, 1].astype(jnp.int32)
    return _sc_mf(uidx, iidx, user_embedding, item_embedding)
